# baseline (speedup 1.0000x reference)
import functools

import jax
import jax.numpy as jnp
from jax import lax
from jax.experimental import pallas as pl
from jax.experimental.pallas import tpu as pltpu
from jax.experimental.pallas import tpu_sc as plsc

HIDDEN = 768
NC = 2
NS = 16
NW = NC * NS
LANES = 16
CHUNK = 32
NBUF = 3


def _zero_pad_rows(idx_v, buf, start, zeros16):
  mm = idx_v[pl.ds(start, LANES)]
  for grp in range(1, CHUNK // LANES):
    mm = jnp.minimum(mm, idx_v[pl.ds(start + grp * LANES, LANES)])
  m = mm[0]
  for lane in range(1, LANES):
    m = jnp.minimum(m, mm[lane])

  @pl.when(m == 0)
  def _():
    for grp in range(CHUNK // LANES):
      v = idx_v[pl.ds(start + grp * LANES, LANES)]
      for lane in range(LANES):
        row = grp * LANES + lane

        @pl.when(v[lane] == 0)
        def _(row=row):
          def zcol(j, carry):
            buf[row, pl.ds(j * LANES, LANES)] = zeros16
            return carry

          lax.fori_loop(0, HIDDEN // LANES, zcol, jnp.int32(0))


def _make_gather(B):
  b_per_w = B // NW
  nchunks = b_per_w // CHUNK
  mesh = plsc.VectorSubcoreMesh(core_axis_name="c", subcore_axis_name="s")

  @functools.partial(
      pl.kernel,
      mesh=mesh,
      out_type=jax.ShapeDtypeStruct((B, HIDDEN), jnp.float32),
      scratch_types=[
          pltpu.VMEM((b_per_w,), jnp.int32),
          pltpu.VMEM((CHUNK, HIDDEN), jnp.float32),
          pltpu.VMEM((CHUNK, HIDDEN), jnp.float32),
          pltpu.VMEM((CHUNK, HIDDEN), jnp.float32),
          pltpu.SemaphoreType.DMA,
          pltpu.SemaphoreType.DMA,
          pltpu.SemaphoreType.DMA,
          pltpu.SemaphoreType.DMA,
          pltpu.SemaphoreType.DMA,
          pltpu.SemaphoreType.DMA,
      ],
  )
  def gather_kernel(table_hbm, idx_hbm, out_hbm, idx_v, buf0, buf1, buf2,
                    gsem0, gsem1, gsem2, ssem0, ssem1, ssem2):
    wid = lax.axis_index("s") * NC + lax.axis_index("c")
    base = wid * b_per_w
    bufs = (buf0, buf1, buf2)
    gsems = (gsem0, gsem1, gsem2)
    ssems = (ssem0, ssem1, ssem2)
    zeros16 = jnp.zeros((LANES,), jnp.float32)

    pltpu.sync_copy(idx_hbm.at[pl.ds(base, b_per_w)], idx_v)

    def gather_copy(g, b):
      return pltpu.make_async_copy(
          table_hbm.at[idx_v.at[pl.ds(g * CHUNK, CHUNK)]], bufs[b],
          gsems[b])

    def scatter_copy(g, b):
      return pltpu.make_async_copy(
          bufs[b], out_hbm.at[pl.ds(base + g * CHUNK, CHUNK)], ssems[b])

    def process(g, b, first=False, prefetch=True):
      gather_copy(g, b).wait()
      _zero_pad_rows(idx_v, bufs[b], g * CHUNK, zeros16)
      scatter_copy(g, b).start()
      if prefetch:
        nb = (b + 2) % NBUF
        if not first:
          scatter_copy(g - 1, nb).wait()
        gather_copy(g + 2, nb).start()

    gather_copy(0, 0).start()
    gather_copy(1, 1).start()
    process(0, 0, first=True)

    def outer(i, carry):
      g = 3 * i + 1
      process(g, 1)
      process(g + 1, 2)
      process(g + 2, 0)
      return carry

    n_steady = (nchunks - 3) // 3
    tail_start = 1 + 3 * n_steady
    lax.fori_loop(0, n_steady, outer, jnp.int32(0))
    for g in range(tail_start, nchunks):
      process(g, g % NBUF, prefetch=(g + 2 < nchunks))
    for g in range(nchunks - 3, nchunks):
      scatter_copy(g, g % NBUF).wait()

  return gather_kernel


_gather = _make_gather(1024 * 200)


def kernel(content_idxs, bert_word_embed):
  idx = content_idxs.reshape(-1).astype(jnp.int32)
  out = _gather(bert_word_embed.astype(jnp.float32), idx)
  return out.reshape(content_idxs.shape + (HIDDEN,))


# speedup vs baseline: 1.0181x; 1.0181x over previous
import functools

import jax
import jax.numpy as jnp
from jax import lax
from jax.experimental import pallas as pl
from jax.experimental.pallas import tpu as pltpu
from jax.experimental.pallas import tpu_sc as plsc

HIDDEN = 768
NC = 2
NS = 16
NW = NC * NS
LANES = 16
CHUNK = 32
NBUF = 4


def _zero_pad_rows(idx_v, buf, start, zeros16):
  mm = idx_v[pl.ds(start, LANES)]
  for grp in range(1, CHUNK // LANES):
    mm = jnp.minimum(mm, idx_v[pl.ds(start + grp * LANES, LANES)])
  m = mm[0]
  for lane in range(1, LANES):
    m = jnp.minimum(m, mm[lane])

  @pl.when(m == 0)
  def _():
    for grp in range(CHUNK // LANES):
      v = idx_v[pl.ds(start + grp * LANES, LANES)]
      for lane in range(LANES):
        row = grp * LANES + lane

        @pl.when(v[lane] == 0)
        def _(row=row):
          def zcol(j, carry):
            buf[row, pl.ds(j * LANES, LANES)] = zeros16
            return carry

          lax.fori_loop(0, HIDDEN // LANES, zcol, jnp.int32(0))


def _make_gather(B):
  b_per_w = B // NW
  nchunks = b_per_w // CHUNK
  mesh = plsc.VectorSubcoreMesh(core_axis_name="c", subcore_axis_name="s")

  @functools.partial(
      pl.kernel,
      mesh=mesh,
      out_type=jax.ShapeDtypeStruct((B, HIDDEN), jnp.float32),
      scratch_types=(
          [pltpu.VMEM((b_per_w,), jnp.int32)]
          + [pltpu.VMEM((CHUNK, HIDDEN), jnp.float32)] * NBUF
          + [pltpu.SemaphoreType.DMA] * (2 * NBUF)
      ),
  )
  def gather_kernel(table_hbm, idx_hbm, out_hbm, idx_v, *scratch):
    wid = lax.axis_index("s") * NC + lax.axis_index("c")
    base = wid * b_per_w
    bufs = scratch[:NBUF]
    gsems = scratch[NBUF:2 * NBUF]
    ssems = scratch[2 * NBUF:]
    zeros16 = jnp.zeros((LANES,), jnp.float32)

    pltpu.sync_copy(idx_hbm.at[pl.ds(base, b_per_w)], idx_v)

    def gather_copy(g, b):
      return pltpu.make_async_copy(
          table_hbm.at[idx_v.at[pl.ds(g * CHUNK, CHUNK)]], bufs[b],
          gsems[b])

    def scatter_copy(g, b):
      return pltpu.make_async_copy(
          bufs[b], out_hbm.at[pl.ds(base + g * CHUNK, CHUNK)], ssems[b])

    def process(g, b, first=False, prefetch=True):
      gather_copy(g, b).wait()
      _zero_pad_rows(idx_v, bufs[b], g * CHUNK, zeros16)
      scatter_copy(g, b).start()
      if prefetch:
        nb = (b + NBUF - 1) % NBUF
        if not first:
          scatter_copy(g - 1, nb).wait()
        gather_copy(g + NBUF - 1, nb).start()

    for b in range(NBUF - 1):
      gather_copy(b, b).start()
    process(0, 0, first=True)

    def outer(i, carry):
      g0 = NBUF * i + 1
      for k in range(NBUF):
        process(g0 + k, (1 + k) % NBUF)
      return carry

    n_steady = (nchunks - NBUF) // NBUF
    tail_start = 1 + NBUF * n_steady
    lax.fori_loop(0, n_steady, outer, jnp.int32(0))
    for g in range(tail_start, nchunks):
      process(g, g % NBUF, prefetch=(g + NBUF - 1 < nchunks))
    for g in range(nchunks - NBUF, nchunks):
      scatter_copy(g, g % NBUF).wait()

  return gather_kernel


_gather = _make_gather(1024 * 200)


def kernel(content_idxs, bert_word_embed):
  idx = content_idxs.reshape(-1).astype(jnp.int32)
  out = _gather(bert_word_embed.astype(jnp.float32), idx)
  return out.reshape(content_idxs.shape + (HIDDEN,))
